# baseline (speedup 1.0000x reference)
import jax
import jax.numpy as jnp
from jax import lax
from jax.experimental import pallas as pl
from jax.experimental.pallas import tpu as pltpu
from jax.experimental.pallas import tpu_sc as plsc

N_NODES = 10000
N_EDGES = 320000
D = 128
NQ = 256
TOK = 1000
NKEEP = N_NODES - TOK

NC = 2
NS = 16
B = 40
ROWS_PER_TILE = N_NODES // NS

EBLK = 6400
LO_BLKS = 22
HI_BLKS = 28
LO_EDGES = LO_BLKS * EBLK
HI_EDGES = HI_BLKS * EBLK
NB_LO = LO_EDGES // (NC * NS) // B
NB_HI = HI_EDGES // (NC * NS) // B


def _xm_body(x_ref, wm_ref, b_ref, o_ref):
    o_ref[...] = (
        jnp.dot(x_ref[...], wm_ref[...], preferred_element_type=jnp.float32)
        + b_ref[...]
    )


def _ea_body(ea_ref, we_ref, o_ref):
    o_ref[...] = lax.dot_general(
        ea_ref[...], we_ref[...], (((0,), (0,)), ((), ())),
        preferred_element_type=jnp.float32,
    )


def _make_sc_body(nbatch, gbase):
    def _sc_body(src_ref, dst_ref, xm_ref, ea_ref, out_ref,
                 xr0, xr1, er0, er1, or0, or1, si0, si1, di0, di1, aggS,
                 sld0, sld1, ssc0, ssc1, ssi0, ssi1, sdi0, sdi1):
        c = lax.axis_index("c")
        s = lax.axis_index("s")
        wid = c * NS + s
        ebase = wid * (nbatch * B)
        ibase = gbase + ebase

        zeros16 = jnp.zeros((16,), jnp.float32)

        def zrow(r, carry):
            for ch in range(8):
                or0[r, pl.ds(ch * 16, 16)] = zeros16
            return carry

        lax.fori_loop(0, B, zrow, 0)
        for k in range(15):
            pltpu.sync_copy(or0, aggS.at[pl.ds(s * ROWS_PER_TILE + k * B, B), :])
        pltpu.sync_copy(
            or0.at[pl.ds(0, 25)],
            aggS.at[pl.ds(s * ROWS_PER_TILE + 600, 25), :],
        )
        plsc.subcore_barrier()

        slot0 = (xr0, er0, or0, si0, di0, sld0, ssc0, ssi0, sdi0)
        slot1 = (xr1, er1, or1, si1, di1, sld1, ssc1, ssi1, sdi1)

        def issue_loads(slot, i):
            xr, er, _, si, _, sld, _, _, _ = slot
            pltpu.async_copy(xm_ref.at[si], xr, sld)
            pltpu.async_copy(ea_ref.at[pl.ds(ebase + i * B, B), :], er, sld)

        def handle(slot, i, first, pref):
            xr, er, orb, si, di, sld, ssc, ssi, sdi = slot
            pltpu.make_async_copy(xm_ref.at[si], xr, sld).wait()
            pltpu.make_async_copy(ea_ref.at[pl.ds(0, B), :], er, sld).wait()
            if pref is not None:
                pltpu.async_copy(src_ref.at[pl.ds(ibase + pref * B, B)], si, ssi)
            if not first:
                pltpu.make_async_copy(orb, aggS.at[di], ssc).wait()
            pltpu.async_copy(dst_ref.at[pl.ds(ibase + i * B, B)], di, sdi)

            def rowfn(r, rc):
                for ch in range(8):
                    sl = pl.ds(ch * 16, 16)
                    orb[r, sl] = jnp.maximum(xr[r, sl] + er[r, sl], 0.0)
                return rc

            lax.fori_loop(0, B, rowfn, 0)
            pltpu.make_async_copy(dst_ref.at[pl.ds(0, B)], di, sdi).wait()
            pltpu.async_copy(orb, aggS.at[di], ssc, add=True)
            if pref is not None:
                pltpu.make_async_copy(src_ref.at[pl.ds(0, B)], si, ssi).wait()
                issue_loads(slot, pref)

        pltpu.sync_copy(src_ref.at[pl.ds(ibase, B)], si0)
        pltpu.sync_copy(src_ref.at[pl.ds(ibase + B, B)], si1)
        issue_loads(slot0, 0)
        issue_loads(slot1, 1)
        handle(slot0, 0, True, 2)
        handle(slot1, 1, True, 3)

        def gbody(g, carry):
            handle(slot0, 2 * g, False, 2 * g + 2)
            handle(slot1, 2 * g + 1, False, 2 * g + 3)
            return carry

        lax.fori_loop(1, nbatch // 2 - 1, gbody, 0)
        handle(slot0, nbatch - 2, False, None)
        handle(slot1, nbatch - 1, False, None)
        pltpu.make_async_copy(or0, aggS.at[di0], ssc0).wait()
        pltpu.make_async_copy(or1, aggS.at[di1], ssc1).wait()
        plsc.subcore_barrier()
        pltpu.sync_copy(
            aggS.at[pl.ds(s * 624, 624), :],
            out_ref.at[c, pl.ds(s * 624, 624), :],
        )

        @pl.when(s == NS - 1)
        def _tail():
            pltpu.sync_copy(
                aggS.at[pl.ds(9984, 16), :],
                out_ref.at[c, pl.ds(9984, 16), :],
            )

    return _sc_body


def _fin_body(agg_a_ref, agg_b_ref, x_ref, wu_ref, ws_ref, q_ref, tn_ref,
              xh_ref, prob_ref):
    agg = (agg_a_ref[0] + agg_a_ref[1]) + (agg_b_ref[0] + agg_b_ref[1])
    xh = jnp.maximum(
        jnp.dot(agg, wu_ref[...], preferred_element_type=jnp.float32)
        + jnp.dot(x_ref[...], ws_ref[...], preferred_element_type=jnp.float32),
        0.0,
    )
    xh_ref[...] = xh
    tn = tn_ref[0]

    def qstep(i, acc):
        return acc + xh_ref[pl.ds(q_ref[i] + tn, 1), :]

    qsum = lax.fori_loop(0, NQ, qstep, jnp.zeros((1, D), jnp.float32))
    qbar = qsum * (1.0 / NQ)
    slab = xh_ref[pl.ds(tn, NKEEP), :]
    p2 = lax.dot_general(
        slab, qbar, (((1,), (1,)), ((), ())), preferred_element_type=jnp.float32
    )
    prob_ref[...] = p2[:, 0]


_SC_SCRATCH = [
    pltpu.VMEM((B, D), jnp.float32),
    pltpu.VMEM((B, D), jnp.float32),
    pltpu.VMEM((B, D), jnp.float32),
    pltpu.VMEM((B, D), jnp.float32),
    pltpu.VMEM((B, D), jnp.float32),
    pltpu.VMEM((B, D), jnp.float32),
    pltpu.VMEM((B,), jnp.int32),
    pltpu.VMEM((B,), jnp.int32),
    pltpu.VMEM((B,), jnp.int32),
    pltpu.VMEM((B,), jnp.int32),
    pltpu.VMEM_SHARED((N_NODES, D), jnp.float32),
] + [pltpu.SemaphoreType.DMA] * 8


def kernel(x, edge_index, edge_attr, query, token_num, W_msg, W_edge, b_msg, W_upd, W_self):
    xm = pl.pallas_call(
        _xm_body,
        out_shape=jax.ShapeDtypeStruct((N_NODES, D), jnp.float32),
    )(x, W_msg, b_msg.reshape(1, D))

    ea_t = edge_attr.T
    ea_lo = pl.pallas_call(
        _ea_body,
        grid=(LO_BLKS,),
        in_specs=[
            pl.BlockSpec((16, EBLK), lambda g: (0, g)),
            pl.BlockSpec((16, D), lambda g: (0, 0)),
        ],
        out_specs=pl.BlockSpec((EBLK, D), lambda g: (g, 0)),
        out_shape=jax.ShapeDtypeStruct((LO_EDGES, D), jnp.float32),
    )(ea_t, W_edge)
    ea_hi = pl.pallas_call(
        _ea_body,
        grid=(HI_BLKS,),
        in_specs=[
            pl.BlockSpec((16, EBLK), lambda g: (0, LO_BLKS + g)),
            pl.BlockSpec((16, D), lambda g: (0, 0)),
        ],
        out_specs=pl.BlockSpec((EBLK, D), lambda g: (g, 0)),
        out_shape=jax.ShapeDtypeStruct((HI_EDGES, D), jnp.float32),
    )(ea_t, W_edge)

    mesh = plsc.VectorSubcoreMesh(core_axis_name="c", subcore_axis_name="s")
    agg_lo = pl.kernel(
        _make_sc_body(NB_LO, 0),
        out_type=jax.ShapeDtypeStruct((NC, N_NODES, D), jnp.float32),
        mesh=mesh,
        scratch_types=_SC_SCRATCH,
    )(edge_index[0], edge_index[1], xm, ea_lo)
    agg_hi = pl.kernel(
        _make_sc_body(NB_HI, LO_EDGES),
        out_type=jax.ShapeDtypeStruct((NC, N_NODES, D), jnp.float32),
        mesh=mesh,
        scratch_types=_SC_SCRATCH,
    )(edge_index[0], edge_index[1], xm, ea_hi)

    tn_arr = jnp.reshape(token_num, (1,)).astype(jnp.int32)
    x_hid, prob2d = pl.pallas_call(
        _fin_body,
        in_specs=[
            pl.BlockSpec(memory_space=pltpu.VMEM),
            pl.BlockSpec(memory_space=pltpu.VMEM),
            pl.BlockSpec(memory_space=pltpu.VMEM),
            pl.BlockSpec(memory_space=pltpu.VMEM),
            pl.BlockSpec(memory_space=pltpu.VMEM),
            pl.BlockSpec(memory_space=pltpu.SMEM),
            pl.BlockSpec(memory_space=pltpu.SMEM),
        ],
        out_shape=(
            jax.ShapeDtypeStruct((N_NODES, D), jnp.float32),
            jax.ShapeDtypeStruct((NKEEP,), jnp.float32),
        ),
    )(agg_lo, agg_hi, x, W_upd, W_self, query, tn_arr)

    return (prob2d, x_hid)


# speedup vs baseline: 1.0021x; 1.0021x over previous
import jax
import jax.numpy as jnp
from jax import lax
from jax.experimental import pallas as pl
from jax.experimental.pallas import tpu as pltpu
from jax.experimental.pallas import tpu_sc as plsc

N_NODES = 10000
N_EDGES = 320000
D = 128
NQ = 256
TOK = 1000
NKEEP = N_NODES - TOK

NC = 2
NS = 16
B = 40
ROWS_PER_TILE = N_NODES // NS

EBLK = 6400
LO_BLKS = 26
HI_BLKS = 24
LO_EDGES = LO_BLKS * EBLK
HI_EDGES = HI_BLKS * EBLK
NB_LO = LO_EDGES // (NC * NS) // B
NB_HI = HI_EDGES // (NC * NS) // B


def _xm_body(x_ref, wm_ref, b_ref, o_ref):
    o_ref[...] = (
        jnp.dot(x_ref[...], wm_ref[...], preferred_element_type=jnp.float32)
        + b_ref[...]
    )


def _ea_body(ea_ref, we_ref, o_ref):
    o_ref[...] = lax.dot_general(
        ea_ref[...], we_ref[...], (((0,), (0,)), ((), ())),
        preferred_element_type=jnp.float32,
    )


def _make_sc_body(nbatch, gbase):
    def _sc_body(src_ref, dst_ref, xm_ref, ea_ref, out_ref,
                 xr0, xr1, er0, er1, or0, or1, si0, si1, di0, di1, aggS,
                 sld0, sld1, ssc0, ssc1, ssi0, ssi1, sdi0, sdi1):
        c = lax.axis_index("c")
        s = lax.axis_index("s")
        wid = c * NS + s
        ebase = wid * (nbatch * B)
        ibase = gbase + ebase

        zeros16 = jnp.zeros((16,), jnp.float32)

        def zrow(r, carry):
            for ch in range(8):
                or0[r, pl.ds(ch * 16, 16)] = zeros16
            return carry

        lax.fori_loop(0, B, zrow, 0)
        for k in range(15):
            pltpu.sync_copy(or0, aggS.at[pl.ds(s * ROWS_PER_TILE + k * B, B), :])
        pltpu.sync_copy(
            or0.at[pl.ds(0, 25)],
            aggS.at[pl.ds(s * ROWS_PER_TILE + 600, 25), :],
        )
        plsc.subcore_barrier()

        slot0 = (xr0, er0, or0, si0, di0, sld0, ssc0, ssi0, sdi0)
        slot1 = (xr1, er1, or1, si1, di1, sld1, ssc1, ssi1, sdi1)

        def issue_loads(slot, i):
            xr, er, _, si, _, sld, _, _, _ = slot
            pltpu.async_copy(xm_ref.at[si], xr, sld)
            pltpu.async_copy(ea_ref.at[pl.ds(ebase + i * B, B), :], er, sld)

        def handle(slot, i, first, pref):
            xr, er, orb, si, di, sld, ssc, ssi, sdi = slot
            pltpu.make_async_copy(xm_ref.at[si], xr, sld).wait()
            pltpu.make_async_copy(ea_ref.at[pl.ds(0, B), :], er, sld).wait()
            if pref is not None:
                pltpu.async_copy(src_ref.at[pl.ds(ibase + pref * B, B)], si, ssi)
            if not first:
                pltpu.make_async_copy(orb, aggS.at[di], ssc).wait()
            pltpu.async_copy(dst_ref.at[pl.ds(ibase + i * B, B)], di, sdi)

            def rowfn(r, rc):
                for ch in range(8):
                    sl = pl.ds(ch * 16, 16)
                    orb[r, sl] = jnp.maximum(xr[r, sl] + er[r, sl], 0.0)
                return rc

            lax.fori_loop(0, B, rowfn, 0)
            pltpu.make_async_copy(dst_ref.at[pl.ds(0, B)], di, sdi).wait()
            pltpu.async_copy(orb, aggS.at[di], ssc, add=True)
            if pref is not None:
                pltpu.make_async_copy(src_ref.at[pl.ds(0, B)], si, ssi).wait()
                issue_loads(slot, pref)

        pltpu.sync_copy(src_ref.at[pl.ds(ibase, B)], si0)
        pltpu.sync_copy(src_ref.at[pl.ds(ibase + B, B)], si1)
        issue_loads(slot0, 0)
        issue_loads(slot1, 1)
        handle(slot0, 0, True, 2)
        handle(slot1, 1, True, 3)

        def gbody(g, carry):
            handle(slot0, 2 * g, False, 2 * g + 2)
            handle(slot1, 2 * g + 1, False, 2 * g + 3)
            return carry

        lax.fori_loop(1, nbatch // 2 - 1, gbody, 0)
        handle(slot0, nbatch - 2, False, None)
        handle(slot1, nbatch - 1, False, None)
        pltpu.make_async_copy(or0, aggS.at[di0], ssc0).wait()
        pltpu.make_async_copy(or1, aggS.at[di1], ssc1).wait()
        plsc.subcore_barrier()
        pltpu.sync_copy(
            aggS.at[pl.ds(s * 624, 624), :],
            out_ref.at[c, pl.ds(s * 624, 624), :],
        )

        @pl.when(s == NS - 1)
        def _tail():
            pltpu.sync_copy(
                aggS.at[pl.ds(9984, 16), :],
                out_ref.at[c, pl.ds(9984, 16), :],
            )

    return _sc_body


def _fin_body(agg_a_ref, agg_b_ref, x_ref, wu_ref, ws_ref, q_ref, tn_ref,
              xh_ref, prob_ref):
    agg = (agg_a_ref[0] + agg_a_ref[1]) + (agg_b_ref[0] + agg_b_ref[1])
    xh = jnp.maximum(
        jnp.dot(agg, wu_ref[...], preferred_element_type=jnp.float32)
        + jnp.dot(x_ref[...], ws_ref[...], preferred_element_type=jnp.float32),
        0.0,
    )
    xh_ref[...] = xh
    tn = tn_ref[0]

    def qstep(i, acc):
        return acc + xh_ref[pl.ds(q_ref[i] + tn, 1), :]

    qsum = lax.fori_loop(0, NQ, qstep, jnp.zeros((1, D), jnp.float32))
    qbar = qsum * (1.0 / NQ)
    slab = xh_ref[pl.ds(tn, NKEEP), :]
    p2 = lax.dot_general(
        slab, qbar, (((1,), (1,)), ((), ())), preferred_element_type=jnp.float32
    )
    prob_ref[...] = p2[:, 0]


_SC_SCRATCH = [
    pltpu.VMEM((B, D), jnp.float32),
    pltpu.VMEM((B, D), jnp.float32),
    pltpu.VMEM((B, D), jnp.float32),
    pltpu.VMEM((B, D), jnp.float32),
    pltpu.VMEM((B, D), jnp.float32),
    pltpu.VMEM((B, D), jnp.float32),
    pltpu.VMEM((B,), jnp.int32),
    pltpu.VMEM((B,), jnp.int32),
    pltpu.VMEM((B,), jnp.int32),
    pltpu.VMEM((B,), jnp.int32),
    pltpu.VMEM_SHARED((N_NODES, D), jnp.float32),
] + [pltpu.SemaphoreType.DMA] * 8


def kernel(x, edge_index, edge_attr, query, token_num, W_msg, W_edge, b_msg, W_upd, W_self):
    xm = pl.pallas_call(
        _xm_body,
        out_shape=jax.ShapeDtypeStruct((N_NODES, D), jnp.float32),
    )(x, W_msg, b_msg.reshape(1, D))

    ea_t = edge_attr.T
    ea_lo = pl.pallas_call(
        _ea_body,
        grid=(LO_BLKS,),
        in_specs=[
            pl.BlockSpec((16, EBLK), lambda g: (0, g)),
            pl.BlockSpec((16, D), lambda g: (0, 0)),
        ],
        out_specs=pl.BlockSpec((EBLK, D), lambda g: (g, 0)),
        out_shape=jax.ShapeDtypeStruct((LO_EDGES, D), jnp.float32),
    )(ea_t, W_edge)
    ea_hi = pl.pallas_call(
        _ea_body,
        grid=(HI_BLKS,),
        in_specs=[
            pl.BlockSpec((16, EBLK), lambda g: (0, LO_BLKS + g)),
            pl.BlockSpec((16, D), lambda g: (0, 0)),
        ],
        out_specs=pl.BlockSpec((EBLK, D), lambda g: (g, 0)),
        out_shape=jax.ShapeDtypeStruct((HI_EDGES, D), jnp.float32),
    )(ea_t, W_edge)

    mesh = plsc.VectorSubcoreMesh(core_axis_name="c", subcore_axis_name="s")
    agg_lo = pl.kernel(
        _make_sc_body(NB_LO, 0),
        out_type=jax.ShapeDtypeStruct((NC, N_NODES, D), jnp.float32),
        mesh=mesh,
        scratch_types=_SC_SCRATCH,
    )(edge_index[0], edge_index[1], xm, ea_lo)
    agg_hi = pl.kernel(
        _make_sc_body(NB_HI, LO_EDGES),
        out_type=jax.ShapeDtypeStruct((NC, N_NODES, D), jnp.float32),
        mesh=mesh,
        scratch_types=_SC_SCRATCH,
    )(edge_index[0], edge_index[1], xm, ea_hi)

    tn_arr = jnp.reshape(token_num, (1,)).astype(jnp.int32)
    x_hid, prob2d = pl.pallas_call(
        _fin_body,
        in_specs=[
            pl.BlockSpec(memory_space=pltpu.VMEM),
            pl.BlockSpec(memory_space=pltpu.VMEM),
            pl.BlockSpec(memory_space=pltpu.VMEM),
            pl.BlockSpec(memory_space=pltpu.VMEM),
            pl.BlockSpec(memory_space=pltpu.VMEM),
            pl.BlockSpec(memory_space=pltpu.SMEM),
            pl.BlockSpec(memory_space=pltpu.SMEM),
        ],
        out_shape=(
            jax.ShapeDtypeStruct((N_NODES, D), jnp.float32),
            jax.ShapeDtypeStruct((NKEEP,), jnp.float32),
        ),
    )(agg_lo, agg_hi, x, W_upd, W_self, query, tn_arr)

    return (prob2d, x_hid)


# speedup vs baseline: 1.0044x; 1.0023x over previous
import jax
import jax.numpy as jnp
from jax import lax
from jax.experimental import pallas as pl
from jax.experimental.pallas import tpu as pltpu
from jax.experimental.pallas import tpu_sc as plsc

N_NODES = 10000
N_EDGES = 320000
D = 128
NQ = 256
TOK = 1000
NKEEP = N_NODES - TOK

NC = 2
NS = 16
B = 40
ROWS_PER_TILE = N_NODES // NS

EBLK = 6400
LO_BLKS = 24
HI_BLKS = 26
LO_EDGES = LO_BLKS * EBLK
HI_EDGES = HI_BLKS * EBLK
NB_LO = LO_EDGES // (NC * NS) // B
NB_HI = HI_EDGES // (NC * NS) // B


def _xm_body(x_ref, wm_ref, b_ref, o_ref):
    o_ref[...] = (
        jnp.dot(x_ref[...], wm_ref[...], preferred_element_type=jnp.float32)
        + b_ref[...]
    )


def _ea_body(ea_ref, we_ref, o_ref):
    o_ref[...] = lax.dot_general(
        ea_ref[...], we_ref[...], (((0,), (0,)), ((), ())),
        preferred_element_type=jnp.float32,
    )


def _make_sc_body(nbatch, gbase):
    def _sc_body(src_ref, dst_ref, xm_ref, ea_ref, out_ref,
                 xr0, xr1, er0, er1, or0, or1, si0, si1, di0, di1, aggS,
                 sld0, sld1, ssc0, ssc1, ssi0, ssi1, sdi0, sdi1):
        c = lax.axis_index("c")
        s = lax.axis_index("s")
        wid = c * NS + s
        ebase = wid * (nbatch * B)
        ibase = gbase + ebase

        zeros16 = jnp.zeros((16,), jnp.float32)

        def zrow(r, carry):
            for ch in range(8):
                or0[r, pl.ds(ch * 16, 16)] = zeros16
            return carry

        lax.fori_loop(0, B, zrow, 0)
        for k in range(15):
            pltpu.sync_copy(or0, aggS.at[pl.ds(s * ROWS_PER_TILE + k * B, B), :])
        pltpu.sync_copy(
            or0.at[pl.ds(0, 25)],
            aggS.at[pl.ds(s * ROWS_PER_TILE + 600, 25), :],
        )
        plsc.subcore_barrier()

        slot0 = (xr0, er0, or0, si0, di0, sld0, ssc0, ssi0, sdi0)
        slot1 = (xr1, er1, or1, si1, di1, sld1, ssc1, ssi1, sdi1)

        def issue_loads(slot, i):
            xr, er, _, si, _, sld, _, _, _ = slot
            pltpu.async_copy(xm_ref.at[si], xr, sld)
            pltpu.async_copy(ea_ref.at[pl.ds(ebase + i * B, B), :], er, sld)

        def handle(slot, i, first, pref):
            xr, er, orb, si, di, sld, ssc, ssi, sdi = slot
            pltpu.make_async_copy(xm_ref.at[si], xr, sld).wait()
            pltpu.make_async_copy(ea_ref.at[pl.ds(0, B), :], er, sld).wait()
            if pref is not None:
                pltpu.async_copy(src_ref.at[pl.ds(ibase + pref * B, B)], si, ssi)
            if not first:
                pltpu.make_async_copy(orb, aggS.at[di], ssc).wait()
            pltpu.async_copy(dst_ref.at[pl.ds(ibase + i * B, B)], di, sdi)

            def rowfn(r, rc):
                for ch in range(8):
                    sl = pl.ds(ch * 16, 16)
                    orb[r, sl] = jnp.maximum(xr[r, sl] + er[r, sl], 0.0)
                return rc

            lax.fori_loop(0, B, rowfn, 0)
            pltpu.make_async_copy(dst_ref.at[pl.ds(0, B)], di, sdi).wait()
            pltpu.async_copy(orb, aggS.at[di], ssc, add=True)
            if pref is not None:
                pltpu.make_async_copy(src_ref.at[pl.ds(0, B)], si, ssi).wait()
                issue_loads(slot, pref)

        pltpu.sync_copy(src_ref.at[pl.ds(ibase, B)], si0)
        pltpu.sync_copy(src_ref.at[pl.ds(ibase + B, B)], si1)
        issue_loads(slot0, 0)
        issue_loads(slot1, 1)
        handle(slot0, 0, True, 2)
        handle(slot1, 1, True, 3)

        def gbody(g, carry):
            handle(slot0, 2 * g, False, 2 * g + 2)
            handle(slot1, 2 * g + 1, False, 2 * g + 3)
            return carry

        lax.fori_loop(1, nbatch // 2 - 1, gbody, 0)
        handle(slot0, nbatch - 2, False, None)
        handle(slot1, nbatch - 1, False, None)
        pltpu.make_async_copy(or0, aggS.at[di0], ssc0).wait()
        pltpu.make_async_copy(or1, aggS.at[di1], ssc1).wait()
        plsc.subcore_barrier()
        pltpu.sync_copy(
            aggS.at[pl.ds(s * 624, 624), :],
            out_ref.at[c, pl.ds(s * 624, 624), :],
        )

        @pl.when(s == NS - 1)
        def _tail():
            pltpu.sync_copy(
                aggS.at[pl.ds(9984, 16), :],
                out_ref.at[c, pl.ds(9984, 16), :],
            )

    return _sc_body


def _fin_body(agg_a_ref, agg_b_ref, x_ref, wu_ref, ws_ref, q_ref, tn_ref,
              xh_ref, prob_ref):
    agg = (agg_a_ref[0] + agg_a_ref[1]) + (agg_b_ref[0] + agg_b_ref[1])
    xh = jnp.maximum(
        jnp.dot(agg, wu_ref[...], preferred_element_type=jnp.float32)
        + jnp.dot(x_ref[...], ws_ref[...], preferred_element_type=jnp.float32),
        0.0,
    )
    xh_ref[...] = xh
    tn = tn_ref[0]

    def qstep(i, acc):
        return acc + xh_ref[pl.ds(q_ref[i] + tn, 1), :]

    qsum = lax.fori_loop(0, NQ, qstep, jnp.zeros((1, D), jnp.float32))
    qbar = qsum * (1.0 / NQ)
    slab = xh_ref[pl.ds(tn, NKEEP), :]
    p2 = lax.dot_general(
        slab, qbar, (((1,), (1,)), ((), ())), preferred_element_type=jnp.float32
    )
    prob_ref[...] = p2[:, 0]


_SC_SCRATCH = [
    pltpu.VMEM((B, D), jnp.float32),
    pltpu.VMEM((B, D), jnp.float32),
    pltpu.VMEM((B, D), jnp.float32),
    pltpu.VMEM((B, D), jnp.float32),
    pltpu.VMEM((B, D), jnp.float32),
    pltpu.VMEM((B, D), jnp.float32),
    pltpu.VMEM((B,), jnp.int32),
    pltpu.VMEM((B,), jnp.int32),
    pltpu.VMEM((B,), jnp.int32),
    pltpu.VMEM((B,), jnp.int32),
    pltpu.VMEM_SHARED((N_NODES, D), jnp.float32),
] + [pltpu.SemaphoreType.DMA] * 8


def kernel(x, edge_index, edge_attr, query, token_num, W_msg, W_edge, b_msg, W_upd, W_self):
    xm = pl.pallas_call(
        _xm_body,
        out_shape=jax.ShapeDtypeStruct((N_NODES, D), jnp.float32),
    )(x, W_msg, b_msg.reshape(1, D))

    ea_t = edge_attr.T
    ea_lo = pl.pallas_call(
        _ea_body,
        grid=(LO_BLKS,),
        in_specs=[
            pl.BlockSpec((16, EBLK), lambda g: (0, g)),
            pl.BlockSpec((16, D), lambda g: (0, 0)),
        ],
        out_specs=pl.BlockSpec((EBLK, D), lambda g: (g, 0)),
        out_shape=jax.ShapeDtypeStruct((LO_EDGES, D), jnp.float32),
    )(ea_t, W_edge)
    ea_hi = pl.pallas_call(
        _ea_body,
        grid=(HI_BLKS,),
        in_specs=[
            pl.BlockSpec((16, EBLK), lambda g: (0, LO_BLKS + g)),
            pl.BlockSpec((16, D), lambda g: (0, 0)),
        ],
        out_specs=pl.BlockSpec((EBLK, D), lambda g: (g, 0)),
        out_shape=jax.ShapeDtypeStruct((HI_EDGES, D), jnp.float32),
    )(ea_t, W_edge)

    mesh = plsc.VectorSubcoreMesh(core_axis_name="c", subcore_axis_name="s")
    agg_lo = pl.kernel(
        _make_sc_body(NB_LO, 0),
        out_type=jax.ShapeDtypeStruct((NC, N_NODES, D), jnp.float32),
        mesh=mesh,
        scratch_types=_SC_SCRATCH,
    )(edge_index[0], edge_index[1], xm, ea_lo)
    agg_hi = pl.kernel(
        _make_sc_body(NB_HI, LO_EDGES),
        out_type=jax.ShapeDtypeStruct((NC, N_NODES, D), jnp.float32),
        mesh=mesh,
        scratch_types=_SC_SCRATCH,
    )(edge_index[0], edge_index[1], xm, ea_hi)

    tn_arr = jnp.reshape(token_num, (1,)).astype(jnp.int32)
    x_hid, prob2d = pl.pallas_call(
        _fin_body,
        in_specs=[
            pl.BlockSpec(memory_space=pltpu.VMEM),
            pl.BlockSpec(memory_space=pltpu.VMEM),
            pl.BlockSpec(memory_space=pltpu.VMEM),
            pl.BlockSpec(memory_space=pltpu.VMEM),
            pl.BlockSpec(memory_space=pltpu.VMEM),
            pl.BlockSpec(memory_space=pltpu.SMEM),
            pl.BlockSpec(memory_space=pltpu.SMEM),
        ],
        out_shape=(
            jax.ShapeDtypeStruct((N_NODES, D), jnp.float32),
            jax.ShapeDtypeStruct((NKEEP,), jnp.float32),
        ),
    )(agg_lo, agg_hi, x, W_upd, W_self, query, tn_arr)

    return (prob2d, x_hid)


# speedup vs baseline: 1.0070x; 1.0026x over previous
import jax
import jax.numpy as jnp
from jax import lax
from jax.experimental import pallas as pl
from jax.experimental.pallas import tpu as pltpu
from jax.experimental.pallas import tpu_sc as plsc

N_NODES = 10000
N_EDGES = 320000
D = 128
NQ = 256
TOK = 1000
NKEEP = N_NODES - TOK

NC = 2
NS = 16
B = 40
ROWS_PER_TILE = N_NODES // NS

EBLK = 6400
LO_BLKS = 24
HI_BLKS = 26
LO_EDGES = LO_BLKS * EBLK
HI_EDGES = HI_BLKS * EBLK
NB_LO = LO_EDGES // (NC * NS) // B
NB_HI = HI_EDGES // (NC * NS) // B


def _xm_body(x_ref, wm_ref, b_ref, o_ref):
    o_ref[...] = (
        jnp.dot(x_ref[...], wm_ref[...], preferred_element_type=jnp.float32)
        + b_ref[...]
    )


def _ea_body(ea_ref, we_ref, o_ref):
    o_ref[...] = lax.dot_general(
        ea_ref[...], we_ref[...], (((0,), (0,)), ((), ())),
        preferred_element_type=jnp.float32,
    )


def _ea_body_after_xm(ea_ref, we_ref, xm_ref, o_ref):
    o_ref[...] = lax.dot_general(
        ea_ref[...], we_ref[...], (((0,), (0,)), ((), ())),
        preferred_element_type=jnp.float32,
    ) + 0.0 * xm_ref[0, 0]


def _make_sc_body(nbatch, gbase):
    def _sc_body(src_ref, dst_ref, xm_ref, ea_ref, out_ref,
                 xr0, xr1, er0, er1, or0, or1, si0, si1, di0, di1, aggS,
                 sld0, sld1, ssc0, ssc1, ssi0, ssi1, sdi0, sdi1):
        c = lax.axis_index("c")
        s = lax.axis_index("s")
        wid = c * NS + s
        ebase = wid * (nbatch * B)
        ibase = gbase + ebase

        zeros16 = jnp.zeros((16,), jnp.float32)

        def zrow(r, carry):
            for ch in range(8):
                or0[r, pl.ds(ch * 16, 16)] = zeros16
            return carry

        lax.fori_loop(0, B, zrow, 0)
        for k in range(15):
            pltpu.sync_copy(or0, aggS.at[pl.ds(s * ROWS_PER_TILE + k * B, B), :])
        pltpu.sync_copy(
            or0.at[pl.ds(0, 25)],
            aggS.at[pl.ds(s * ROWS_PER_TILE + 600, 25), :],
        )
        plsc.subcore_barrier()

        slot0 = (xr0, er0, or0, si0, di0, sld0, ssc0, ssi0, sdi0)
        slot1 = (xr1, er1, or1, si1, di1, sld1, ssc1, ssi1, sdi1)

        def issue_loads(slot, i):
            xr, er, _, si, _, sld, _, _, _ = slot
            pltpu.async_copy(xm_ref.at[si], xr, sld)
            pltpu.async_copy(ea_ref.at[pl.ds(ebase + i * B, B), :], er, sld)

        def handle(slot, i, first, pref):
            xr, er, orb, si, di, sld, ssc, ssi, sdi = slot
            pltpu.make_async_copy(xm_ref.at[si], xr, sld).wait()
            pltpu.make_async_copy(ea_ref.at[pl.ds(0, B), :], er, sld).wait()
            if pref is not None:
                pltpu.async_copy(src_ref.at[pl.ds(ibase + pref * B, B)], si, ssi)
            if not first:
                pltpu.make_async_copy(orb, aggS.at[di], ssc).wait()
            pltpu.async_copy(dst_ref.at[pl.ds(ibase + i * B, B)], di, sdi)

            def rowfn(r, rc):
                for ch in range(8):
                    sl = pl.ds(ch * 16, 16)
                    orb[r, sl] = jnp.maximum(xr[r, sl] + er[r, sl], 0.0)
                return rc

            lax.fori_loop(0, B, rowfn, 0)
            pltpu.make_async_copy(dst_ref.at[pl.ds(0, B)], di, sdi).wait()
            pltpu.async_copy(orb, aggS.at[di], ssc, add=True)
            if pref is not None:
                pltpu.make_async_copy(src_ref.at[pl.ds(0, B)], si, ssi).wait()
                issue_loads(slot, pref)

        pltpu.sync_copy(src_ref.at[pl.ds(ibase, B)], si0)
        pltpu.sync_copy(src_ref.at[pl.ds(ibase + B, B)], si1)
        issue_loads(slot0, 0)
        issue_loads(slot1, 1)
        handle(slot0, 0, True, 2)
        handle(slot1, 1, True, 3)

        def gbody(g, carry):
            handle(slot0, 2 * g, False, 2 * g + 2)
            handle(slot1, 2 * g + 1, False, 2 * g + 3)
            return carry

        lax.fori_loop(1, nbatch // 2 - 1, gbody, 0)
        handle(slot0, nbatch - 2, False, None)
        handle(slot1, nbatch - 1, False, None)
        pltpu.make_async_copy(or0, aggS.at[di0], ssc0).wait()
        pltpu.make_async_copy(or1, aggS.at[di1], ssc1).wait()
        plsc.subcore_barrier()
        pltpu.sync_copy(
            aggS.at[pl.ds(s * 624, 624), :],
            out_ref.at[c, pl.ds(s * 624, 624), :],
        )

        @pl.when(s == NS - 1)
        def _tail():
            pltpu.sync_copy(
                aggS.at[pl.ds(9984, 16), :],
                out_ref.at[c, pl.ds(9984, 16), :],
            )

    return _sc_body


def _fin_body(agg_a_ref, agg_b_ref, x_ref, wu_ref, ws_ref, q_ref, tn_ref,
              xh_ref, prob_ref):
    agg = (agg_a_ref[0] + agg_a_ref[1]) + (agg_b_ref[0] + agg_b_ref[1])
    xh = jnp.maximum(
        jnp.dot(agg, wu_ref[...], preferred_element_type=jnp.float32)
        + jnp.dot(x_ref[...], ws_ref[...], preferred_element_type=jnp.float32),
        0.0,
    )
    xh_ref[...] = xh
    tn = tn_ref[0]

    def qstep(i, acc):
        return acc + xh_ref[pl.ds(q_ref[i] + tn, 1), :]

    qsum = lax.fori_loop(0, NQ, qstep, jnp.zeros((1, D), jnp.float32))
    qbar = qsum * (1.0 / NQ)
    slab = xh_ref[pl.ds(tn, NKEEP), :]
    p2 = lax.dot_general(
        slab, qbar, (((1,), (1,)), ((), ())), preferred_element_type=jnp.float32
    )
    prob_ref[...] = p2[:, 0]


_SC_SCRATCH = [
    pltpu.VMEM((B, D), jnp.float32),
    pltpu.VMEM((B, D), jnp.float32),
    pltpu.VMEM((B, D), jnp.float32),
    pltpu.VMEM((B, D), jnp.float32),
    pltpu.VMEM((B, D), jnp.float32),
    pltpu.VMEM((B, D), jnp.float32),
    pltpu.VMEM((B,), jnp.int32),
    pltpu.VMEM((B,), jnp.int32),
    pltpu.VMEM((B,), jnp.int32),
    pltpu.VMEM((B,), jnp.int32),
    pltpu.VMEM_SHARED((N_NODES, D), jnp.float32),
] + [pltpu.SemaphoreType.DMA] * 8


def kernel(x, edge_index, edge_attr, query, token_num, W_msg, W_edge, b_msg, W_upd, W_self):
    xm = pl.pallas_call(
        _xm_body,
        out_shape=jax.ShapeDtypeStruct((N_NODES, D), jnp.float32),
    )(x, W_msg, b_msg.reshape(1, D))

    ea_t = edge_attr.T
    ea_lo = pl.pallas_call(
        _ea_body_after_xm,
        grid=(LO_BLKS,),
        in_specs=[
            pl.BlockSpec((16, EBLK), lambda g: (0, g)),
            pl.BlockSpec((16, D), lambda g: (0, 0)),
            pl.BlockSpec((8, D), lambda g: (0, 0)),
        ],
        out_specs=pl.BlockSpec((EBLK, D), lambda g: (g, 0)),
        out_shape=jax.ShapeDtypeStruct((LO_EDGES, D), jnp.float32),
    )(ea_t, W_edge, xm)
    ea_hi = pl.pallas_call(
        _ea_body,
        grid=(HI_BLKS,),
        in_specs=[
            pl.BlockSpec((16, EBLK), lambda g: (0, LO_BLKS + g)),
            pl.BlockSpec((16, D), lambda g: (0, 0)),
        ],
        out_specs=pl.BlockSpec((EBLK, D), lambda g: (g, 0)),
        out_shape=jax.ShapeDtypeStruct((HI_EDGES, D), jnp.float32),
    )(ea_t, W_edge)

    mesh = plsc.VectorSubcoreMesh(core_axis_name="c", subcore_axis_name="s")
    agg_lo = pl.kernel(
        _make_sc_body(NB_LO, 0),
        out_type=jax.ShapeDtypeStruct((NC, N_NODES, D), jnp.float32),
        mesh=mesh,
        scratch_types=_SC_SCRATCH,
    )(edge_index[0], edge_index[1], xm, ea_lo)
    agg_hi = pl.kernel(
        _make_sc_body(NB_HI, LO_EDGES),
        out_type=jax.ShapeDtypeStruct((NC, N_NODES, D), jnp.float32),
        mesh=mesh,
        scratch_types=_SC_SCRATCH,
    )(edge_index[0], edge_index[1], xm, ea_hi)

    tn_arr = jnp.reshape(token_num, (1,)).astype(jnp.int32)
    x_hid, prob2d = pl.pallas_call(
        _fin_body,
        in_specs=[
            pl.BlockSpec(memory_space=pltpu.VMEM),
            pl.BlockSpec(memory_space=pltpu.VMEM),
            pl.BlockSpec(memory_space=pltpu.VMEM),
            pl.BlockSpec(memory_space=pltpu.VMEM),
            pl.BlockSpec(memory_space=pltpu.VMEM),
            pl.BlockSpec(memory_space=pltpu.SMEM),
            pl.BlockSpec(memory_space=pltpu.SMEM),
        ],
        out_shape=(
            jax.ShapeDtypeStruct((N_NODES, D), jnp.float32),
            jax.ShapeDtypeStruct((NKEEP,), jnp.float32),
        ),
    )(agg_lo, agg_hi, x, W_upd, W_self, query, tn_arr)

    return (prob2d, x_hid)
